# Initial kernel scaffold; baseline (speedup 1.0000x reference)
import jax
import jax.numpy as jnp
from jax.experimental import pallas as pl


def kernel(y_pre, y_batch):
    raise NotImplementedError("write your pallas kernel here")


# speedup vs baseline: 3.4474x; 3.4474x over previous
import functools

import jax
import jax.numpy as jnp
from jax import lax
from jax.experimental import pallas as pl

_C = 6
_BETA = 0.5
_B = 32
_A = 65536
_CHUNK = 4096
_NJ = _A // _CHUNK
_INF_BITS = 0x7F800000


def _pass1_body(pre_ref, bat_ref, x_ref, np_ref, pl_ref, bx_ref, nx_ref):
    j = pl.program_id(1)

    pre = pre_ref[0]
    bat = bat_ref[0]
    c_pre = pre[:_C, :]
    b_pre = pre[_C:, :]
    c_hat = bat[:_C, :]
    b_hat = bat[_C:, :]

    mx = jnp.max(c_hat, axis=0, keepdims=True)
    pos = mx > 0.0
    ch_iota = lax.broadcasted_iota(jnp.int32, (_C, _CHUNK), 0).astype(
        jnp.float32)
    first = jnp.min(jnp.where(c_hat == mx, ch_iota, float(_C)),
                    axis=0, keepdims=True)
    tgt_logit = jnp.sum(jnp.where(ch_iota == first, c_pre, 0.0),
                        axis=0, keepdims=True)

    m = jnp.max(c_pre, axis=0, keepdims=True)
    lse = m + jnp.log(jnp.sum(jnp.exp(c_pre - m), axis=0, keepdims=True))

    ce = lse - tgt_logit
    x = lse - c_pre[0:1, :]
    x_ref[0, 0, :] = jnp.where(pos, -1.0, x)[0]

    d = jnp.abs(b_pre - b_hat)
    sl1 = jnp.where(d < 1.0, 0.5 * d * d, d - 0.5)
    box = jnp.sum(jnp.where(pos, sl1, 0.0))

    n_pos = jnp.sum(jnp.where(pos, 1.0, 0.0))
    pos_ce = jnp.sum(jnp.where(pos, ce, 0.0))
    neg_x = jnp.sum(jnp.where(pos, 0.0, x))

    zero = jnp.zeros((1, 1, 1), jnp.float32)

    @pl.when(j == 0)
    def _():
        np_ref[...] = zero
        pl_ref[...] = zero
        bx_ref[...] = zero
        nx_ref[...] = zero

    np_ref[...] += n_pos.reshape(1, 1, 1)
    pl_ref[...] += pos_ce.reshape(1, 1, 1)
    bx_ref[...] += box.reshape(1, 1, 1)
    nx_ref[...] += neg_x.reshape(1, 1, 1)


def _pass2_body(x_ref, np_ref, pl_ref, bx_ref, nx_ref, o1_ref, o2_ref, o3_ref):
    X = x_ref[:, 0, :]
    n_pos = np_ref[...].reshape(_B, 1)
    pos_loss = pl_ref[...].reshape(_B, 1)
    box_sum = bx_ref[...].reshape(_B, 1)
    s_all = nx_ref[...].reshape(_B, 1)

    n_neg = float(_A) - n_pos
    k = jnp.minimum(3.0 * n_pos, n_neg)

    unresolved = jnp.logical_and(k > 0.0, k < n_neg)
    lo0 = jnp.zeros((_B, 1), jnp.int32)
    hi0 = jnp.where(unresolved, _INF_BITS, 0).astype(jnp.int32)

    def cond(carry):
        lo, hi = carry
        return jnp.max(hi - lo) > 0

    def body(carry):
        lo, hi = carry
        mid = lo + ((hi - lo) >> 1)
        t = lax.bitcast_convert_type(mid, jnp.float32)
        cnt = jnp.sum(jnp.where(X > t, 1.0, 0.0), axis=1, keepdims=True)
        p = cnt < k
        return jnp.where(p, lo, mid + 1), jnp.where(p, mid, hi)

    lo, _ = lax.while_loop(cond, body, (lo0, hi0))

    t = lax.bitcast_convert_type(lo, jnp.float32)
    gt = X > t
    cnt_gt = jnp.sum(jnp.where(gt, 1.0, 0.0), axis=1, keepdims=True)
    sum_gt = jnp.sum(jnp.where(gt, X, 0.0), axis=1, keepdims=True)
    neg_sel = sum_gt + (k - cnt_gt) * t

    neg_loss = jnp.where(k <= 0.0, 0.0, jnp.where(k >= n_neg, s_all, neg_sel))

    has_neg = n_neg > 0.0
    denom = jnp.maximum(jnp.where(has_neg, k + n_pos, n_pos), 1.0)
    lpb = jnp.where(has_neg, pos_loss + neg_loss, pos_loss) / denom
    valid = n_pos > 0.0
    count = jnp.sum(jnp.where(valid, 1.0, 0.0))
    class_sum = jnp.sum(jnp.where(valid, lpb, 0.0))
    l_class = jnp.where(count > 0.0, class_sum / jnp.maximum(count, 1.0), 0.0)

    npt = jnp.sum(n_pos)
    l_box = jnp.where(npt > 0.0, jnp.sum(box_sum) / (npt + 1e-6), 0.0)

    o1_ref[...] = (l_class + _BETA * l_box).reshape(1, 1)
    o2_ref[...] = l_class.reshape(1, 1)
    o3_ref[...] = l_box.reshape(1, 1)


@functools.partial(jax.jit, static_argnames=("interpret",))
def kernel(y_pre, y_batch, interpret=False):
    yt_pre = jnp.transpose(y_pre, (0, 2, 1))
    yt_bat = jnp.transpose(y_batch, (0, 2, 1))

    part = jax.ShapeDtypeStruct((_B, 1, 1), jnp.float32)
    X, n_pos, pos_loss, box_sum, neg_x = pl.pallas_call(
        _pass1_body,
        grid=(_B, _NJ),
        in_specs=[
            pl.BlockSpec((1, 10, _CHUNK), lambda b, j: (b, 0, j)),
            pl.BlockSpec((1, 10, _CHUNK), lambda b, j: (b, 0, j)),
        ],
        out_specs=[
            pl.BlockSpec((1, 1, _CHUNK), lambda b, j: (b, 0, j)),
            pl.BlockSpec((1, 1, 1), lambda b, j: (b, 0, 0)),
            pl.BlockSpec((1, 1, 1), lambda b, j: (b, 0, 0)),
            pl.BlockSpec((1, 1, 1), lambda b, j: (b, 0, 0)),
            pl.BlockSpec((1, 1, 1), lambda b, j: (b, 0, 0)),
        ],
        out_shape=[
            jax.ShapeDtypeStruct((_B, 1, _A), jnp.float32),
            part, part, part, part,
        ],
        interpret=interpret,
    )(yt_pre, yt_bat)

    scal = jax.ShapeDtypeStruct((1, 1), jnp.float32)
    total, l_class, l_box = pl.pallas_call(
        _pass2_body,
        out_shape=[scal, scal, scal],
        interpret=interpret,
    )(X, n_pos, pos_loss, box_sum, neg_x)

    return (total[0, 0], l_class[0, 0], l_box[0, 0])


# speedup vs baseline: 4.8426x; 1.4047x over previous
import functools

import jax
import jax.numpy as jnp
from jax import lax
from jax.experimental import pallas as pl

_C = 6
_BETA = 0.5
_B = 32
_A = 65536
_LANES = 128
_ROWS = _A // _LANES
_S = 64
_NJ = _ROWS // _S
_INF_BITS = 0x7F800000


def _pass1_body(pre_ref, bat_ref, x_ref, np_ref, pl_ref, bx_ref, nx_ref):
    j = pl.program_id(1)

    pr = [pre_ref[0, c] for c in range(10)]
    ch = [bat_ref[0, c] for c in range(10)]

    mx = ch[0]
    for c in range(1, _C):
        mx = jnp.maximum(mx, ch[c])
    pos = mx > 0.0

    tgt = pr[_C - 1]
    for c in range(_C - 1, -1, -1):
        tgt = jnp.where(ch[c] == mx, pr[c], tgt)

    m = pr[0]
    for c in range(1, _C):
        m = jnp.maximum(m, pr[c])
    e = jnp.exp(pr[0] - m)
    for c in range(1, _C):
        e = e + jnp.exp(pr[c] - m)
    lse = m + jnp.log(e)

    ce = lse - tgt
    x = lse - pr[0]
    x_ref[0] = jnp.where(pos, -1.0, x)

    sl1 = None
    for c in range(_C, 10):
        d = jnp.abs(pr[c] - ch[c])
        t = jnp.where(d < 1.0, 0.5 * d * d, d - 0.5)
        sl1 = t if sl1 is None else sl1 + t

    n_pos = jnp.sum(jnp.where(pos, 1.0, 0.0))
    pos_ce = jnp.sum(jnp.where(pos, ce, 0.0))
    neg_x = jnp.sum(jnp.where(pos, 0.0, x))
    box = jnp.sum(jnp.where(pos, sl1, 0.0))

    zero = jnp.zeros((1, 1, 1), jnp.float32)

    @pl.when(j == 0)
    def _():
        np_ref[...] = zero
        pl_ref[...] = zero
        bx_ref[...] = zero
        nx_ref[...] = zero

    np_ref[...] += n_pos.reshape(1, 1, 1)
    pl_ref[...] += pos_ce.reshape(1, 1, 1)
    bx_ref[...] += box.reshape(1, 1, 1)
    nx_ref[...] += neg_x.reshape(1, 1, 1)


def _pass2_body(x_ref, np_ref, pl_ref, bx_ref, nx_ref, o1_ref, o2_ref, o3_ref):
    X = x_ref[...]
    n_pos = np_ref[...]
    pos_loss = pl_ref[...]
    box_sum = bx_ref[...]
    s_all = nx_ref[...]

    n_neg = float(_A) - n_pos
    k = jnp.minimum(3.0 * n_pos, n_neg)

    unresolved = jnp.logical_and(k > 0.0, k < n_neg)
    lo0 = jnp.zeros((_B, 1, 1), jnp.int32)
    hi0 = jnp.where(unresolved, _INF_BITS, 0).astype(jnp.int32)

    def cond(carry):
        lo, hi = carry
        return jnp.max(hi - lo) > 0

    def body(carry):
        lo, hi = carry
        mid = lo + ((hi - lo) >> 1)
        t = lax.bitcast_convert_type(mid, jnp.float32)
        cnt = jnp.sum(jnp.where(X > t, 1.0, 0.0), axis=(1, 2), keepdims=True)
        p = cnt < k
        return jnp.where(p, lo, mid + 1), jnp.where(p, mid, hi)

    lo, _ = lax.while_loop(cond, body, (lo0, hi0))

    t = lax.bitcast_convert_type(lo, jnp.float32)
    gt = X > t
    cnt_gt = jnp.sum(jnp.where(gt, 1.0, 0.0), axis=(1, 2), keepdims=True)
    sum_gt = jnp.sum(jnp.where(gt, X, 0.0), axis=(1, 2), keepdims=True)
    neg_sel = sum_gt + (k - cnt_gt) * t

    neg_loss = jnp.where(k <= 0.0, 0.0, jnp.where(k >= n_neg, s_all, neg_sel))

    has_neg = n_neg > 0.0
    denom = jnp.maximum(jnp.where(has_neg, k + n_pos, n_pos), 1.0)
    lpb = jnp.where(has_neg, pos_loss + neg_loss, pos_loss) / denom
    valid = n_pos > 0.0
    count = jnp.sum(jnp.where(valid, 1.0, 0.0))
    class_sum = jnp.sum(jnp.where(valid, lpb, 0.0))
    l_class = jnp.where(count > 0.0, class_sum / jnp.maximum(count, 1.0), 0.0)

    npt = jnp.sum(n_pos)
    l_box = jnp.where(npt > 0.0, jnp.sum(box_sum) / (npt + 1e-6), 0.0)

    o1_ref[...] = (l_class + _BETA * l_box).reshape(1, 1)
    o2_ref[...] = l_class.reshape(1, 1)
    o3_ref[...] = l_box.reshape(1, 1)


@functools.partial(jax.jit, static_argnames=("interpret",))
def kernel(y_pre, y_batch, interpret=False):
    yt_pre = jnp.transpose(y_pre, (0, 2, 1)).reshape(_B, 10, _ROWS, _LANES)
    yt_bat = jnp.transpose(y_batch, (0, 2, 1)).reshape(_B, 10, _ROWS, _LANES)

    part = jax.ShapeDtypeStruct((_B, 1, 1), jnp.float32)
    X, n_pos, pos_loss, box_sum, neg_x = pl.pallas_call(
        _pass1_body,
        grid=(_B, _NJ),
        in_specs=[
            pl.BlockSpec((1, 10, _S, _LANES), lambda b, j: (b, 0, j, 0)),
            pl.BlockSpec((1, 10, _S, _LANES), lambda b, j: (b, 0, j, 0)),
        ],
        out_specs=[
            pl.BlockSpec((1, _S, _LANES), lambda b, j: (b, j, 0)),
            pl.BlockSpec((1, 1, 1), lambda b, j: (b, 0, 0)),
            pl.BlockSpec((1, 1, 1), lambda b, j: (b, 0, 0)),
            pl.BlockSpec((1, 1, 1), lambda b, j: (b, 0, 0)),
            pl.BlockSpec((1, 1, 1), lambda b, j: (b, 0, 0)),
        ],
        out_shape=[
            jax.ShapeDtypeStruct((_B, _ROWS, _LANES), jnp.float32),
            part, part, part, part,
        ],
        interpret=interpret,
    )(yt_pre, yt_bat)

    scal = jax.ShapeDtypeStruct((1, 1), jnp.float32)
    total, l_class, l_box = pl.pallas_call(
        _pass2_body,
        out_shape=[scal, scal, scal],
        interpret=interpret,
    )(X, n_pos, pos_loss, box_sum, neg_x)

    return (total[0, 0], l_class[0, 0], l_box[0, 0])
